# Initial kernel scaffold; baseline (speedup 1.0000x reference)
import jax
import jax.numpy as jnp
from jax.experimental import pallas as pl


def kernel(x, edge_index, batch, W1, b1, W2, b2, W3, b3, Wc1, bc1, Wc2, bc2):
    raise NotImplementedError("write your pallas kernel here")


# speedup vs baseline: 1.2986x; 1.2986x over previous
import functools

import jax
import jax.numpy as jnp
from jax.experimental import pallas as pl

N = 10000
NPAD = 10240


def _mm_body(x_ref, w_ref, o_ref):
    o_ref[...] = jnp.dot(x_ref[...], w_ref[...], preferred_element_type=jnp.float32)


def _matmul(x, w):
    m, k = x.shape
    k2, n = w.shape
    bm = 1024
    mp = ((m + bm - 1) // bm) * bm
    if mp != m:
        x = jnp.pad(x, ((0, mp - m), (0, 0)))
    out = pl.pallas_call(
        _mm_body,
        grid=(mp // bm,),
        in_specs=[
            pl.BlockSpec((bm, k), lambda i: (i, 0)),
            pl.BlockSpec((k, n), lambda i: (0, 0)),
        ],
        out_specs=pl.BlockSpec((bm, n), lambda i: (i, 0)),
        out_shape=jax.ShapeDtypeStruct((mp, n), jnp.float32),
    )(x, w)
    return out[:m]


def _gcn(x, src, dst, norm, self_coef, W, b):
    h = _matmul(x, W)
    msg = h[src] * norm[:, None]
    out = jnp.zeros((N, W.shape[1]), jnp.float32).at[dst].add(msg)
    out = out + h * self_coef[:, None]
    return jax.nn.relu(out + b)


def kernel(x, edge_index, batch, W1, b1, W2, b2, W3, b3, Wc1, bc1, Wc2, bc2):
    src, dst = edge_index[0], edge_index[1]
    deg = jnp.ones((N,), jnp.float32).at[dst].add(1.0)
    dinv = jax.lax.rsqrt(deg)
    norm = dinv[src] * dinv[dst]
    self_coef = dinv * dinv

    h = _gcn(x, src, dst, norm, self_coef, W1, b1)
    h = _gcn(h, src, dst, norm, self_coef, W2, b2)
    h = _gcn(h, src, dst, norm, self_coef, W3, b3)

    sums = jax.ops.segment_sum(h, batch, num_segments=64)
    counts = jax.ops.segment_sum(jnp.ones((N, 1), jnp.float32), batch, num_segments=64)
    pooled = sums / jnp.maximum(counts, 1.0)
    z = jax.nn.relu(pooled @ Wc1 + bc1)
    z = z @ Wc2 + bc2
    return jax.nn.sigmoid(z)


# speedup vs baseline: 8.3702x; 6.4454x over previous
import functools

import jax
import jax.numpy as jnp
from jax import lax
from jax.experimental import pallas as pl
from jax.experimental.pallas import tpu as pltpu
from jax.experimental.pallas import tpu_sc as plsc

N = 10000
NPAD = 10240
E = 320000
B = 64
CHUNK = 80
EROWS = E // CHUNK
TILES = 16
NODES_PER_TILE = NPAD // TILES

_mesh = plsc.VectorSubcoreMesh(core_axis_name="c", subcore_axis_name="s")


DEGW = 128


def _deg_body(dst_flat, z16, ones16, out0, out1, acc, dst_v, ones_v, sem):
    c = lax.axis_index("c")
    s = lax.axis_index("s")
    epw = E // (2 * TILES)
    sl = pl.ds(s * NODES_PER_TILE, NODES_PER_TILE)

    pltpu.sync_copy(ones16, ones_v)
    pltpu.sync_copy(z16.at[sl], acc.at[sl])
    plsc.subcore_barrier()

    def body(i, _):
        base = (c * TILES + s) * epw + i * CHUNK
        pltpu.sync_copy(dst_flat.at[pl.ds(base, CHUNK)], dst_v)
        pltpu.sync_copy(ones_v, acc.at[dst_v], add=True)
        return 0

    lax.fori_loop(0, epw // CHUNK, body, 0)
    plsc.subcore_barrier()

    @pl.when(c == 0)
    def _():
        pltpu.sync_copy(acc.at[sl], out0.at[sl])

    @pl.when(c == 1)
    def _():
        pltpu.sync_copy(acc.at[sl], out1.at[sl])


def _deg_call(dst_r):
    z16 = jnp.zeros((NPAD, DEGW), jnp.float32)
    ones16 = jnp.ones((CHUNK, DEGW), jnp.float32)
    f = pl.kernel(
        _deg_body,
        mesh=_mesh,
        out_type=[jax.ShapeDtypeStruct((NPAD, DEGW), jnp.float32),
                  jax.ShapeDtypeStruct((NPAD, DEGW), jnp.float32)],
        scratch_types=[
            pltpu.VMEM_SHARED((NPAD, DEGW), jnp.float32),
            pltpu.VMEM((CHUNK,), jnp.int32),
            pltpu.VMEM((CHUNK, DEGW), jnp.float32),
            pltpu.SemaphoreType.DMA,
        ],
    )
    return f(dst_r, z16, ones16)


def _agg_body(src_flat, dst_flat, g0, g1, o0, o1, acc, src_v, dst_v,
              rows_v, sem):
    c = lax.axis_index("c")
    s = lax.axis_index("s")
    ept = E // TILES
    sl = pl.ds(s * NODES_PER_TILE, NODES_PER_TILE)

    def work(g_hbm, o_hbm):
        pltpu.sync_copy(g_hbm.at[sl], acc.at[sl])
        plsc.subcore_barrier()

        def body(i, _):
            base = s * ept + i * CHUNK
            pltpu.sync_copy(src_flat.at[pl.ds(base, CHUNK)], src_v)
            pltpu.sync_copy(dst_flat.at[pl.ds(base, CHUNK)], dst_v)
            pltpu.async_copy(g_hbm.at[src_v], rows_v, sem).wait()
            pltpu.sync_copy(rows_v, acc.at[dst_v], add=True)
            return 0

        lax.fori_loop(0, ept // CHUNK, body, 0)
        plsc.subcore_barrier()
        pltpu.sync_copy(acc.at[sl], o_hbm.at[sl])

    @pl.when(c == 0)
    def _():
        work(g0, o0)

    @pl.when(c == 1)
    def _():
        work(g1, o1)


def _agg2_body(src_flat, dst_flat, g, z, o0, o1, acc, src_v, dst_v,
               rows_v, sem):
    c = lax.axis_index("c")
    s = lax.axis_index("s")
    ept = E // (2 * TILES)
    sl = pl.ds(s * NODES_PER_TILE, NODES_PER_TILE)

    def work(init_hbm, o_hbm):
        pltpu.sync_copy(init_hbm.at[sl], acc.at[sl])
        plsc.subcore_barrier()

        def body(i, _):
            base = c * (E // 2) + s * ept + i * CHUNK
            pltpu.sync_copy(src_flat.at[pl.ds(base, CHUNK)], src_v)
            pltpu.sync_copy(dst_flat.at[pl.ds(base, CHUNK)], dst_v)
            pltpu.async_copy(g.at[src_v], rows_v, sem).wait()
            pltpu.sync_copy(rows_v, acc.at[dst_v], add=True)
            return 0

        lax.fori_loop(0, ept // CHUNK, body, 0)
        plsc.subcore_barrier()
        pltpu.sync_copy(acc.at[sl], o_hbm.at[sl])

    @pl.when(c == 0)
    def _():
        work(g, o0)

    @pl.when(c == 1)
    def _():
        work(z, o1)


def _agg2_call(src_r, dst_r, g, z, hc):
    f = pl.kernel(
        _agg2_body,
        mesh=_mesh,
        out_type=[jax.ShapeDtypeStruct((NPAD, hc), jnp.float32),
                  jax.ShapeDtypeStruct((NPAD, hc), jnp.float32)],
        scratch_types=[
            pltpu.VMEM_SHARED((NPAD, hc), jnp.float32),
            pltpu.VMEM((CHUNK,), jnp.int32),
            pltpu.VMEM((CHUNK,), jnp.int32),
            pltpu.VMEM((CHUNK, hc), jnp.float32),
            pltpu.SemaphoreType.DMA,
        ],
    )
    return f(src_r, dst_r, g, z)


def _agg_call(src_r, dst_r, g0, g1, hc):
    f = pl.kernel(
        _agg_body,
        mesh=_mesh,
        out_type=[jax.ShapeDtypeStruct((NPAD, hc), jnp.float32),
                  jax.ShapeDtypeStruct((NPAD, hc), jnp.float32)],
        scratch_types=[
            pltpu.VMEM_SHARED((NPAD, hc), jnp.float32),
            pltpu.VMEM((CHUNK,), jnp.int32),
            pltpu.VMEM((CHUNK,), jnp.int32),
            pltpu.VMEM((CHUNK, hc), jnp.float32),
            pltpu.SemaphoreType.DMA,
        ],
    )
    return f(src_r, dst_r, g0, g1)


_BM = 1024


def _tca_body(x_ref, d0_ref, d1_ref, w_ref, dinv_ref, g0_ref, g1_ref):
    dv = lax.rsqrt(d0_ref[...] + d1_ref[...] + 1.0)
    h = jnp.dot(x_ref[...], w_ref[...], preferred_element_type=jnp.float32)
    g = h * dv
    hh = g.shape[1] // 2
    dinv_ref[...] = dv
    g0_ref[...] = g[:, :hh]
    g1_ref[...] = g[:, hh:]


def _tca(x, deg0, deg1, w):
    h = w.shape[1]
    return pl.pallas_call(
        _tca_body,
        grid=(NPAD // _BM,),
        in_specs=[
            pl.BlockSpec((_BM, x.shape[1]), lambda i: (i, 0)),
            pl.BlockSpec((_BM, 1), lambda i: (i, 0)),
            pl.BlockSpec((_BM, 1), lambda i: (i, 0)),
            pl.BlockSpec(w.shape, lambda i: (0, 0)),
        ],
        out_specs=[
            pl.BlockSpec((_BM, 1), lambda i: (i, 0)),
            pl.BlockSpec((_BM, h // 2), lambda i: (i, 0)),
            pl.BlockSpec((_BM, h // 2), lambda i: (i, 0)),
        ],
        out_shape=[
            jax.ShapeDtypeStruct((NPAD, 1), jnp.float32),
            jax.ShapeDtypeStruct((NPAD, h // 2), jnp.float32),
            jax.ShapeDtypeStruct((NPAD, h // 2), jnp.float32),
        ],
    )(x, deg0, deg1, w)


def _tcb_body(a0_ref, a1_ref, dinv_ref, b_ref, w_ref, g0_ref, g1_ref):
    dv = dinv_ref[...]
    hin = w_ref.shape[0]
    hh = hin // 2
    a0 = jnp.maximum(dv * a0_ref[...] + b_ref[0, :hh], 0.0)
    a1 = jnp.maximum(dv * a1_ref[...] + b_ref[0, hh:], 0.0)
    h = (jnp.dot(a0, w_ref[:hh, :], preferred_element_type=jnp.float32)
         + jnp.dot(a1, w_ref[hh:, :], preferred_element_type=jnp.float32))
    g = h * dv
    ho = g.shape[1] // 2
    g0_ref[...] = g[:, :ho]
    g1_ref[...] = g[:, ho:]


def _tcb(acc0, acc1, dinv, b, w):
    hin, hout = w.shape
    return pl.pallas_call(
        _tcb_body,
        grid=(NPAD // _BM,),
        in_specs=[
            pl.BlockSpec((_BM, hin // 2), lambda i: (i, 0)),
            pl.BlockSpec((_BM, hin // 2), lambda i: (i, 0)),
            pl.BlockSpec((_BM, 1), lambda i: (i, 0)),
            pl.BlockSpec((1, hin), lambda i: (0, 0)),
            pl.BlockSpec((hin, hout), lambda i: (0, 0)),
        ],
        out_specs=[
            pl.BlockSpec((_BM, hout // 2), lambda i: (i, 0)),
            pl.BlockSpec((_BM, hout // 2), lambda i: (i, 0)),
        ],
        out_shape=[
            jax.ShapeDtypeStruct((NPAD, hout // 2), jnp.float32),
            jax.ShapeDtypeStruct((NPAD, hout // 2), jnp.float32),
        ],
    )(acc0, acc1, dinv, b.reshape(1, hin), w)


def _tcb_full_body(a0_ref, a1_ref, dinv_ref, b_ref, w_ref, g_ref):
    dv = dinv_ref[...]
    hin = w_ref.shape[0]
    hh = hin // 2
    a0 = jnp.maximum(dv * a0_ref[...] + b_ref[0, :hh], 0.0)
    a1 = jnp.maximum(dv * a1_ref[...] + b_ref[0, hh:], 0.0)
    h = (jnp.dot(a0, w_ref[:hh, :], preferred_element_type=jnp.float32)
         + jnp.dot(a1, w_ref[hh:, :], preferred_element_type=jnp.float32))
    g_ref[...] = h * dv


def _tcb_full(acc0, acc1, dinv, b, w):
    hin, hout = w.shape
    return pl.pallas_call(
        _tcb_full_body,
        grid=(NPAD // _BM,),
        in_specs=[
            pl.BlockSpec((_BM, hin // 2), lambda i: (i, 0)),
            pl.BlockSpec((_BM, hin // 2), lambda i: (i, 0)),
            pl.BlockSpec((_BM, 1), lambda i: (i, 0)),
            pl.BlockSpec((1, hin), lambda i: (0, 0)),
            pl.BlockSpec((hin, hout), lambda i: (0, 0)),
        ],
        out_specs=pl.BlockSpec((_BM, hout), lambda i: (i, 0)),
        out_shape=jax.ShapeDtypeStruct((NPAD, hout), jnp.float32),
    )(acc0, acc1, dinv, b.reshape(1, hin), w)


def _tcc_body(a0_ref, a1_ref, dinv_ref, b_ref, batch_ref, wc1_ref, bc1_ref,
              wc2_ref, bc2_ref, z_ref, sums_ref, cnts_ref):
    i = pl.program_id(0)
    dv = dinv_ref[...]
    a = jnp.maximum(dv * (a0_ref[...] + a1_ref[...]) + b_ref[0, :], 0.0)
    ids = lax.broadcasted_iota(jnp.int32, (_BM, B), 1)
    p = (batch_ref[...] == ids).astype(jnp.float32)

    @pl.when(i == 0)
    def _():
        sums_ref[...] = jnp.zeros_like(sums_ref)
        cnts_ref[...] = jnp.zeros_like(cnts_ref)

    sums_ref[...] += lax.dot_general(p, a, (((0,), (0,)), ((), ())),
                                     preferred_element_type=jnp.float32)
    cnts_ref[...] += lax.dot_general(p, jnp.ones((_BM, 1), jnp.float32),
                                     (((0,), (0,)), ((), ())),
                                     preferred_element_type=jnp.float32)

    @pl.when(i == pl.num_programs(0) - 1)
    def _():
        pooled = sums_ref[...] / jnp.maximum(cnts_ref[...], 1.0)
        z = jnp.maximum(jnp.dot(pooled, wc1_ref[...],
                                preferred_element_type=jnp.float32)
                        + bc1_ref[0, :], 0.0)
        z = jnp.dot(z, wc2_ref[...], preferred_element_type=jnp.float32) \
            + bc2_ref[0, :]
        z_ref[...] = 1.0 / (1.0 + jnp.exp(-z))


def _tcc(acc0, acc1, dinv, b3, batch2d, wc1, bc1, wc2, bc2):
    hin = acc0.shape[1]
    out = pl.pallas_call(
        _tcc_body,
        grid=(NPAD // _BM,),
        in_specs=[
            pl.BlockSpec((_BM, hin), lambda i: (i, 0)),
            pl.BlockSpec((_BM, hin), lambda i: (i, 0)),
            pl.BlockSpec((_BM, 1), lambda i: (i, 0)),
            pl.BlockSpec((1, hin), lambda i: (0, 0)),
            pl.BlockSpec((_BM, 1), lambda i: (i, 0)),
            pl.BlockSpec(wc1.shape, lambda i: (0, 0)),
            pl.BlockSpec((1, 32), lambda i: (0, 0)),
            pl.BlockSpec(wc2.shape, lambda i: (0, 0)),
            pl.BlockSpec((1, 1), lambda i: (0, 0)),
        ],
        out_specs=[
            pl.BlockSpec((B, 1), lambda i: (0, 0)),
            pl.BlockSpec((B, hin), lambda i: (0, 0)),
            pl.BlockSpec((B, 1), lambda i: (0, 0)),
        ],
        out_shape=[
            jax.ShapeDtypeStruct((B, 1), jnp.float32),
            jax.ShapeDtypeStruct((B, hin), jnp.float32),
            jax.ShapeDtypeStruct((B, 1), jnp.float32),
        ],
    )(acc0, acc1, dinv, b3.reshape(1, hin), batch2d,
      wc1, bc1.reshape(1, 32), wc2, bc2.reshape(1, 1))
    return out[0]


def kernel(x, edge_index, batch, W1, b1, W2, b2, W3, b3, Wc1, bc1, Wc2, bc2):
    src_r = edge_index[0]
    dst_r = edge_index[1]
    xp = jnp.pad(x, ((0, NPAD - N), (0, 0)))
    batch2d = jnp.pad(batch, (0, NPAD - N), constant_values=B).reshape(NPAD, 1)

    deg0, deg1 = _deg_call(dst_r)
    dinv, g0, g1 = _tca(xp, deg0[:, :1], deg1[:, :1], W1)

    o0, o1 = _agg_call(src_r, dst_r, g0, g1, W1.shape[1] // 2)
    g0, g1 = _tcb(o0, o1, dinv, b1, W2)

    o0, o1 = _agg_call(src_r, dst_r, g0, g1, W2.shape[1] // 2)
    g3 = _tcb_full(o0, o1, dinv, b2, W3)

    z = jnp.zeros((NPAD, W3.shape[1]), jnp.float32)
    o0, o1 = _agg2_call(src_r, dst_r, g3, z, W3.shape[1])
    return _tcc(o0, o1, dinv, b3, batch2d, Wc1, bc1, Wc2, bc2)


# speedup vs baseline: 14.0904x; 1.6834x over previous
import functools

import jax
import jax.numpy as jnp
from jax import lax
from jax.experimental import pallas as pl
from jax.experimental.pallas import tpu as pltpu
from jax.experimental.pallas import tpu_sc as plsc

N = 10000
NPAD = 10240
E = 320000
B = 64
CHUNK = 80
EROWS = E // CHUNK
TILES = 16
NODES_PER_TILE = NPAD // TILES

_mesh = plsc.VectorSubcoreMesh(core_axis_name="c", subcore_axis_name="s")


DEGW = 128


def _deg_body(dst_flat, z16, ones16, out0, out1, acc, dst_v, ones_v, sem):
    c = lax.axis_index("c")
    s = lax.axis_index("s")
    epw = E // (2 * TILES)
    sl = pl.ds(s * NODES_PER_TILE, NODES_PER_TILE)

    pltpu.sync_copy(ones16, ones_v)
    pltpu.sync_copy(z16.at[sl], acc.at[sl])
    plsc.subcore_barrier()

    def body(i, _):
        base = (c * TILES + s) * epw + i * CHUNK
        pltpu.sync_copy(dst_flat.at[pl.ds(base, CHUNK)], dst_v)
        pltpu.sync_copy(ones_v, acc.at[dst_v], add=True)
        return 0

    lax.fori_loop(0, epw // CHUNK, body, 0)
    plsc.subcore_barrier()

    @pl.when(c == 0)
    def _():
        pltpu.sync_copy(acc.at[sl], out0.at[sl])

    @pl.when(c == 1)
    def _():
        pltpu.sync_copy(acc.at[sl], out1.at[sl])


def _deg_call(dst_r):
    z16 = jnp.zeros((NPAD, DEGW), jnp.float32)
    ones16 = jnp.ones((CHUNK, DEGW), jnp.float32)
    f = pl.kernel(
        _deg_body,
        mesh=_mesh,
        out_type=[jax.ShapeDtypeStruct((NPAD, DEGW), jnp.float32),
                  jax.ShapeDtypeStruct((NPAD, DEGW), jnp.float32)],
        scratch_types=[
            pltpu.VMEM_SHARED((NPAD, DEGW), jnp.float32),
            pltpu.VMEM((CHUNK,), jnp.int32),
            pltpu.VMEM((CHUNK, DEGW), jnp.float32),
            pltpu.SemaphoreType.DMA,
        ],
    )
    return f(dst_r, z16, ones16)


def _agg_body(src_flat, dst_flat, g0, g1, o0, o1, acc, src_fblk, dst_fblk,
              dst2d, rows2, sem_a, sem_b):
    c = lax.axis_index("c")
    s = lax.axis_index("s")
    ept = E // TILES
    sl = pl.ds(s * NODES_PER_TILE, NODES_PER_TILE)

    def work(g_hbm, o_hbm):
        pltpu.sync_copy(g_hbm.at[sl], acc.at[sl])
        plsc.subcore_barrier()
        _edge_pipeline(s * ept, ept // BLKE, g_hbm, acc, src_fblk, dst_fblk,
                       dst2d, rows2, sem_a, sem_b, src_flat, dst_flat)
        plsc.subcore_barrier()
        pltpu.sync_copy(acc.at[sl], o_hbm.at[sl])

    @pl.when(c == 0)
    def _():
        work(g0, o0)

    @pl.when(c == 1)
    def _():
        work(g1, o1)


NBLK = 5
BLKE = NBLK * CHUNK


def _edge_pipeline(tile_base, nblocks, g_hbm, acc, src_fblk, dst_fblk,
                   dst2d, rows2, sem_a, sem_b, src_flat, dst_flat):
    sems = (sem_a, sem_b)

    def blk_body(b, _):
        base = tile_base + b * BLKE
        pltpu.sync_copy(src_flat.at[pl.ds(base, BLKE)], src_fblk)
        pltpu.sync_copy(dst_flat.at[pl.ds(base, BLKE)], dst_fblk)
        for j in range(NBLK):
            for k in range(CHUNK // 16):
                dst2d[j, pl.ds(k * 16, 16)] = dst_fblk[
                    pl.ds(j * CHUNK + k * 16, 16)]
        cps = [None, None]
        cps[0] = pltpu.async_copy(
            g_hbm.at[src_fblk.at[pl.ds(0, CHUNK)]], rows2.at[0], sems[0])
        for j in range(NBLK):
            p = j & 1
            if j + 1 < NBLK:
                q = (j + 1) & 1
                cps[q] = pltpu.async_copy(
                    g_hbm.at[src_fblk.at[pl.ds((j + 1) * CHUNK, CHUNK)]],
                    rows2.at[q], sems[q])
            cps[p].wait()
            pltpu.sync_copy(rows2.at[p], acc.at[dst2d.at[j]], add=True)
        return 0

    lax.fori_loop(0, nblocks, blk_body, 0)


def _agg2_body(src_flat, dst_flat, g, z, o0, o1, acc, src_fblk, dst_fblk,
               dst2d, rows2, sem_a, sem_b):
    c = lax.axis_index("c")
    s = lax.axis_index("s")
    ept = E // (2 * TILES)
    sl = pl.ds(s * NODES_PER_TILE, NODES_PER_TILE)

    def work(init_hbm, o_hbm):
        pltpu.sync_copy(init_hbm.at[sl], acc.at[sl])
        plsc.subcore_barrier()
        _edge_pipeline(c * (E // 2) + s * ept, ept // BLKE, g, acc,
                       src_fblk, dst_fblk, dst2d, rows2, sem_a, sem_b,
                       src_flat, dst_flat)
        plsc.subcore_barrier()
        pltpu.sync_copy(acc.at[sl], o_hbm.at[sl])

    @pl.when(c == 0)
    def _():
        work(g, o0)

    @pl.when(c == 1)
    def _():
        work(z, o1)


def _pipe_scratch(hc):
    return [
        pltpu.VMEM_SHARED((NPAD, hc), jnp.float32),
        pltpu.VMEM((BLKE,), jnp.int32),
        pltpu.VMEM((BLKE,), jnp.int32),
        pltpu.VMEM((NBLK, CHUNK), jnp.int32),
        pltpu.VMEM((2, CHUNK, hc), jnp.float32),
        pltpu.SemaphoreType.DMA,
        pltpu.SemaphoreType.DMA,
    ]


def _agg2_call(src_r, dst_r, g, z, hc):
    f = pl.kernel(
        _agg2_body,
        mesh=_mesh,
        out_type=[jax.ShapeDtypeStruct((NPAD, hc), jnp.float32),
                  jax.ShapeDtypeStruct((NPAD, hc), jnp.float32)],
        scratch_types=_pipe_scratch(hc),
    )
    return f(src_r, dst_r, g, z)


def _agg_call(src_r, dst_r, g0, g1, hc):
    f = pl.kernel(
        _agg_body,
        mesh=_mesh,
        out_type=[jax.ShapeDtypeStruct((NPAD, hc), jnp.float32),
                  jax.ShapeDtypeStruct((NPAD, hc), jnp.float32)],
        scratch_types=_pipe_scratch(hc),
    )
    return f(src_r, dst_r, g0, g1)


_BM = 1024


def _tca_body(x_ref, d0_ref, d1_ref, w_ref, dinv_ref, g0_ref, g1_ref):
    dv = lax.rsqrt(d0_ref[...] + d1_ref[...] + 1.0)
    h = jnp.dot(x_ref[...], w_ref[...], preferred_element_type=jnp.float32)
    g = h * dv
    hh = g.shape[1] // 2
    dinv_ref[...] = dv
    g0_ref[...] = g[:, :hh]
    g1_ref[...] = g[:, hh:]


def _tca(x, deg0, deg1, w):
    h = w.shape[1]
    return pl.pallas_call(
        _tca_body,
        grid=(NPAD // _BM,),
        in_specs=[
            pl.BlockSpec((_BM, x.shape[1]), lambda i: (i, 0)),
            pl.BlockSpec((_BM, 1), lambda i: (i, 0)),
            pl.BlockSpec((_BM, 1), lambda i: (i, 0)),
            pl.BlockSpec(w.shape, lambda i: (0, 0)),
        ],
        out_specs=[
            pl.BlockSpec((_BM, 1), lambda i: (i, 0)),
            pl.BlockSpec((_BM, h // 2), lambda i: (i, 0)),
            pl.BlockSpec((_BM, h // 2), lambda i: (i, 0)),
        ],
        out_shape=[
            jax.ShapeDtypeStruct((NPAD, 1), jnp.float32),
            jax.ShapeDtypeStruct((NPAD, h // 2), jnp.float32),
            jax.ShapeDtypeStruct((NPAD, h // 2), jnp.float32),
        ],
    )(x, deg0, deg1, w)


def _tcb_body(a0_ref, a1_ref, dinv_ref, b_ref, w_ref, g0_ref, g1_ref):
    dv = dinv_ref[...]
    hin = w_ref.shape[0]
    hh = hin // 2
    a0 = jnp.maximum(dv * a0_ref[...] + b_ref[0, :hh], 0.0)
    a1 = jnp.maximum(dv * a1_ref[...] + b_ref[0, hh:], 0.0)
    h = (jnp.dot(a0, w_ref[:hh, :], preferred_element_type=jnp.float32)
         + jnp.dot(a1, w_ref[hh:, :], preferred_element_type=jnp.float32))
    g = h * dv
    ho = g.shape[1] // 2
    g0_ref[...] = g[:, :ho]
    g1_ref[...] = g[:, ho:]


def _tcb(acc0, acc1, dinv, b, w):
    hin, hout = w.shape
    return pl.pallas_call(
        _tcb_body,
        grid=(NPAD // _BM,),
        in_specs=[
            pl.BlockSpec((_BM, hin // 2), lambda i: (i, 0)),
            pl.BlockSpec((_BM, hin // 2), lambda i: (i, 0)),
            pl.BlockSpec((_BM, 1), lambda i: (i, 0)),
            pl.BlockSpec((1, hin), lambda i: (0, 0)),
            pl.BlockSpec((hin, hout), lambda i: (0, 0)),
        ],
        out_specs=[
            pl.BlockSpec((_BM, hout // 2), lambda i: (i, 0)),
            pl.BlockSpec((_BM, hout // 2), lambda i: (i, 0)),
        ],
        out_shape=[
            jax.ShapeDtypeStruct((NPAD, hout // 2), jnp.float32),
            jax.ShapeDtypeStruct((NPAD, hout // 2), jnp.float32),
        ],
    )(acc0, acc1, dinv, b.reshape(1, hin), w)


def _tcb_full_body(a0_ref, a1_ref, dinv_ref, b_ref, w_ref, g_ref):
    dv = dinv_ref[...]
    hin = w_ref.shape[0]
    hh = hin // 2
    a0 = jnp.maximum(dv * a0_ref[...] + b_ref[0, :hh], 0.0)
    a1 = jnp.maximum(dv * a1_ref[...] + b_ref[0, hh:], 0.0)
    h = (jnp.dot(a0, w_ref[:hh, :], preferred_element_type=jnp.float32)
         + jnp.dot(a1, w_ref[hh:, :], preferred_element_type=jnp.float32))
    g_ref[...] = h * dv


def _tcb_full(acc0, acc1, dinv, b, w):
    hin, hout = w.shape
    return pl.pallas_call(
        _tcb_full_body,
        grid=(NPAD // _BM,),
        in_specs=[
            pl.BlockSpec((_BM, hin // 2), lambda i: (i, 0)),
            pl.BlockSpec((_BM, hin // 2), lambda i: (i, 0)),
            pl.BlockSpec((_BM, 1), lambda i: (i, 0)),
            pl.BlockSpec((1, hin), lambda i: (0, 0)),
            pl.BlockSpec((hin, hout), lambda i: (0, 0)),
        ],
        out_specs=pl.BlockSpec((_BM, hout), lambda i: (i, 0)),
        out_shape=jax.ShapeDtypeStruct((NPAD, hout), jnp.float32),
    )(acc0, acc1, dinv, b.reshape(1, hin), w)


def _tcc_body(a0_ref, a1_ref, dinv_ref, b_ref, batch_ref, wc1_ref, bc1_ref,
              wc2_ref, bc2_ref, z_ref, sums_ref, cnts_ref):
    i = pl.program_id(0)
    dv = dinv_ref[...]
    a = jnp.maximum(dv * (a0_ref[...] + a1_ref[...]) + b_ref[0, :], 0.0)
    ids = lax.broadcasted_iota(jnp.int32, (_BM, B), 1)
    p = (batch_ref[...] == ids).astype(jnp.float32)

    @pl.when(i == 0)
    def _():
        sums_ref[...] = jnp.zeros_like(sums_ref)
        cnts_ref[...] = jnp.zeros_like(cnts_ref)

    sums_ref[...] += lax.dot_general(p, a, (((0,), (0,)), ((), ())),
                                     preferred_element_type=jnp.float32)
    cnts_ref[...] += lax.dot_general(p, jnp.ones((_BM, 1), jnp.float32),
                                     (((0,), (0,)), ((), ())),
                                     preferred_element_type=jnp.float32)

    @pl.when(i == pl.num_programs(0) - 1)
    def _():
        pooled = sums_ref[...] / jnp.maximum(cnts_ref[...], 1.0)
        z = jnp.maximum(jnp.dot(pooled, wc1_ref[...],
                                preferred_element_type=jnp.float32)
                        + bc1_ref[0, :], 0.0)
        z = jnp.dot(z, wc2_ref[...], preferred_element_type=jnp.float32) \
            + bc2_ref[0, :]
        z_ref[...] = 1.0 / (1.0 + jnp.exp(-z))


def _tcc(acc0, acc1, dinv, b3, batch2d, wc1, bc1, wc2, bc2):
    hin = acc0.shape[1]
    out = pl.pallas_call(
        _tcc_body,
        grid=(NPAD // _BM,),
        in_specs=[
            pl.BlockSpec((_BM, hin), lambda i: (i, 0)),
            pl.BlockSpec((_BM, hin), lambda i: (i, 0)),
            pl.BlockSpec((_BM, 1), lambda i: (i, 0)),
            pl.BlockSpec((1, hin), lambda i: (0, 0)),
            pl.BlockSpec((_BM, 1), lambda i: (i, 0)),
            pl.BlockSpec(wc1.shape, lambda i: (0, 0)),
            pl.BlockSpec((1, 32), lambda i: (0, 0)),
            pl.BlockSpec(wc2.shape, lambda i: (0, 0)),
            pl.BlockSpec((1, 1), lambda i: (0, 0)),
        ],
        out_specs=[
            pl.BlockSpec((B, 1), lambda i: (0, 0)),
            pl.BlockSpec((B, hin), lambda i: (0, 0)),
            pl.BlockSpec((B, 1), lambda i: (0, 0)),
        ],
        out_shape=[
            jax.ShapeDtypeStruct((B, 1), jnp.float32),
            jax.ShapeDtypeStruct((B, hin), jnp.float32),
            jax.ShapeDtypeStruct((B, 1), jnp.float32),
        ],
    )(acc0, acc1, dinv, b3.reshape(1, hin), batch2d,
      wc1, bc1.reshape(1, 32), wc2, bc2.reshape(1, 1))
    return out[0]


def kernel(x, edge_index, batch, W1, b1, W2, b2, W3, b3, Wc1, bc1, Wc2, bc2):
    src_r = edge_index[0]
    dst_r = edge_index[1]
    xp = jnp.pad(x, ((0, NPAD - N), (0, 0)))
    batch2d = jnp.pad(batch, (0, NPAD - N), constant_values=B).reshape(NPAD, 1)

    deg0, deg1 = _deg_call(dst_r)
    dinv, g0, g1 = _tca(xp, deg0[:, :1], deg1[:, :1], W1)

    o0, o1 = _agg_call(src_r, dst_r, g0, g1, W1.shape[1] // 2)
    g0, g1 = _tcb(o0, o1, dinv, b1, W2)

    o0, o1 = _agg_call(src_r, dst_r, g0, g1, W2.shape[1] // 2)
    g3 = _tcb_full(o0, o1, dinv, b2, W3)

    z = jnp.zeros((NPAD, W3.shape[1]), jnp.float32)
    o0, o1 = _agg2_call(src_r, dst_r, g3, z, W3.shape[1])
    return _tcc(o0, o1, dinv, b3, batch2d, Wc1, bc1, Wc2, bc2)


# speedup vs baseline: 15.1146x; 1.0727x over previous
import functools

import jax
import jax.numpy as jnp
from jax import lax
from jax.experimental import pallas as pl
from jax.experimental.pallas import tpu as pltpu
from jax.experimental.pallas import tpu_sc as plsc

N = 10000
NPAD = 10240
E = 320000
B = 64
CHUNK = 80
EROWS = E // CHUNK
TILES = 16
NODES_PER_TILE = NPAD // TILES

_mesh = plsc.VectorSubcoreMesh(core_axis_name="c", subcore_axis_name="s")


DEGW = 128


def _deg_body(dst_flat, z16, ones16, out0, out1, acc, dst_fblk, dst2d,
              ones_v, sem_x2, sem_s0, sem_s1):
    c = lax.axis_index("c")
    s = lax.axis_index("s")
    epw = E // (2 * TILES)
    tile_base = (c * TILES + s) * epw
    nblocks = epw // BLKE
    sl = pl.ds(s * NODES_PER_TILE, NODES_PER_TILE)
    sems = (sem_s0, sem_s1)

    pltpu.sync_copy(ones16, ones_v)
    pltpu.sync_copy(z16.at[sl], acc.at[sl])
    pltpu.async_copy(dst_flat.at[pl.ds(tile_base, BLKE)], dst_fblk, sem_x2)
    plsc.subcore_barrier()

    def blk_body(b, _):
        base = tile_base + b * BLKE
        pltpu.make_async_copy(
            dst_flat.at[pl.ds(base, BLKE)], dst_fblk, sem_x2).wait()
        for j in range(NBLK):
            for k in range(CHUNK // 16):
                dst2d[j, pl.ds(k * 16, 16)] = dst_fblk[
                    pl.ds(j * CHUNK + k * 16, 16)]

        @pl.when(b + 1 < nblocks)
        def _():
            pltpu.async_copy(dst_flat.at[pl.ds(base + BLKE, BLKE)],
                             dst_fblk, sem_x2)

        pends = [None, None]
        for j in range(NBLK):
            p = j & 1
            if pends[p] is not None:
                pends[p].wait()
            pends[p] = pltpu.async_copy(ones_v, acc.at[dst2d.at[j]],
                                        sems[p], add=True)
        for cp in pends:
            if cp is not None:
                cp.wait()
        return 0

    lax.fori_loop(0, nblocks, blk_body, 0)
    plsc.subcore_barrier()

    @pl.when(c == 0)
    def _():
        pltpu.sync_copy(acc.at[sl], out0.at[sl])

    @pl.when(c == 1)
    def _():
        pltpu.sync_copy(acc.at[sl], out1.at[sl])


def _deg_call(dst_r):
    z16 = jnp.zeros((NPAD, DEGW), jnp.float32)
    ones16 = jnp.ones((CHUNK, DEGW), jnp.float32)
    f = pl.kernel(
        _deg_body,
        mesh=_mesh,
        out_type=[jax.ShapeDtypeStruct((NPAD, DEGW), jnp.float32),
                  jax.ShapeDtypeStruct((NPAD, DEGW), jnp.float32)],
        scratch_types=[
            pltpu.VMEM_SHARED((NPAD, DEGW), jnp.float32),
            pltpu.VMEM((BLKE,), jnp.int32),
            pltpu.VMEM((NBLK, CHUNK), jnp.int32),
            pltpu.VMEM((CHUNK, DEGW), jnp.float32),
            pltpu.SemaphoreType.DMA,
            pltpu.SemaphoreType.DMA,
            pltpu.SemaphoreType.DMA,
        ],
    )
    return f(dst_r, z16, ones16)


def _agg_body(src_flat, dst_flat, g0, g1, o0, o1, acc, src_fblk, dst_fblk,
              src2d, dst2d, rows2, sem_x1, sem_x2, sem_g0, sem_g1,
              sem_s0, sem_s1):
    c = lax.axis_index("c")
    s = lax.axis_index("s")
    ept = E // TILES
    sl = pl.ds(s * NODES_PER_TILE, NODES_PER_TILE)

    def work(g_hbm, o_hbm):
        pltpu.sync_copy(g_hbm.at[sl], acc.at[sl])
        plsc.subcore_barrier()
        _edge_pipeline(s * ept, ept // BLKE, g_hbm, acc, src_fblk, dst_fblk,
                       src2d, dst2d, rows2, sem_x1, sem_x2, sem_g0, sem_g1,
                       sem_s0, sem_s1, src_flat, dst_flat)
        plsc.subcore_barrier()
        pltpu.sync_copy(acc.at[sl], o_hbm.at[sl])

    @pl.when(c == 0)
    def _():
        work(g0, o0)

    @pl.when(c == 1)
    def _():
        work(g1, o1)


NBLK = 5
BLKE = NBLK * CHUNK


def _edge_pipeline(tile_base, nblocks, g_hbm, acc, src_fblk, dst_fblk,
                   src2d, dst2d, rows2, sem_x1, sem_x2, sem_g0, sem_g1,
                   sem_s0, sem_s1, src_flat, dst_flat):
    sems_g = (sem_g0, sem_g1)
    sems_s = (sem_s0, sem_s1)

    pltpu.async_copy(src_flat.at[pl.ds(tile_base, BLKE)], src_fblk, sem_x1)
    pltpu.async_copy(dst_flat.at[pl.ds(tile_base, BLKE)], dst_fblk, sem_x2)

    def blk_body(b, _):
        base = tile_base + b * BLKE
        pltpu.make_async_copy(
            src_flat.at[pl.ds(base, BLKE)], src_fblk, sem_x1).wait()
        pltpu.make_async_copy(
            dst_flat.at[pl.ds(base, BLKE)], dst_fblk, sem_x2).wait()
        for j in range(NBLK):
            for k in range(CHUNK // 16):
                src2d[j, pl.ds(k * 16, 16)] = src_fblk[
                    pl.ds(j * CHUNK + k * 16, 16)]
                dst2d[j, pl.ds(k * 16, 16)] = dst_fblk[
                    pl.ds(j * CHUNK + k * 16, 16)]

        @pl.when(b + 1 < nblocks)
        def _():
            nb = base + BLKE
            pltpu.async_copy(src_flat.at[pl.ds(nb, BLKE)], src_fblk, sem_x1)
            pltpu.async_copy(dst_flat.at[pl.ds(nb, BLKE)], dst_fblk, sem_x2)

        cps = [None, None]
        cps[0] = pltpu.async_copy(g_hbm.at[src2d.at[0]], rows2.at[0],
                                  sems_g[0])
        prev_s = None
        for j in range(NBLK):
            p = j & 1
            cps[p].wait()
            if prev_s is not None:
                prev_s.wait()
            s_cp = pltpu.async_copy(rows2.at[p], acc.at[dst2d.at[j]],
                                    sems_s[p], add=True)
            if j + 1 < NBLK:
                q = (j + 1) & 1
                cps[q] = pltpu.async_copy(g_hbm.at[src2d.at[j + 1]],
                                          rows2.at[q], sems_g[q])
            prev_s = s_cp
        prev_s.wait()
        return 0

    lax.fori_loop(0, nblocks, blk_body, 0)


def _agg2_body(src_flat, dst_flat, g, z, o0, o1, acc, src_fblk, dst_fblk,
               src2d, dst2d, rows2, sem_x1, sem_x2, sem_g0, sem_g1,
               sem_s0, sem_s1):
    c = lax.axis_index("c")
    s = lax.axis_index("s")
    ept = E // (2 * TILES)
    sl = pl.ds(s * NODES_PER_TILE, NODES_PER_TILE)

    def work(init_hbm, o_hbm):
        pltpu.sync_copy(init_hbm.at[sl], acc.at[sl])
        plsc.subcore_barrier()
        _edge_pipeline(c * (E // 2) + s * ept, ept // BLKE, g, acc,
                       src_fblk, dst_fblk, src2d, dst2d, rows2,
                       sem_x1, sem_x2, sem_g0, sem_g1, sem_s0, sem_s1,
                       src_flat, dst_flat)
        plsc.subcore_barrier()
        pltpu.sync_copy(acc.at[sl], o_hbm.at[sl])

    @pl.when(c == 0)
    def _():
        work(g, o0)

    @pl.when(c == 1)
    def _():
        work(z, o1)


def _pipe_scratch(hc):
    return [
        pltpu.VMEM_SHARED((NPAD, hc), jnp.float32),
        pltpu.VMEM((BLKE,), jnp.int32),
        pltpu.VMEM((BLKE,), jnp.int32),
        pltpu.VMEM((NBLK, CHUNK), jnp.int32),
        pltpu.VMEM((NBLK, CHUNK), jnp.int32),
        pltpu.VMEM((2, CHUNK, hc), jnp.float32),
        pltpu.SemaphoreType.DMA,
        pltpu.SemaphoreType.DMA,
        pltpu.SemaphoreType.DMA,
        pltpu.SemaphoreType.DMA,
        pltpu.SemaphoreType.DMA,
        pltpu.SemaphoreType.DMA,
    ]


def _agg2_call(src_r, dst_r, g, z, hc):
    f = pl.kernel(
        _agg2_body,
        mesh=_mesh,
        out_type=[jax.ShapeDtypeStruct((NPAD, hc), jnp.float32),
                  jax.ShapeDtypeStruct((NPAD, hc), jnp.float32)],
        scratch_types=_pipe_scratch(hc),
    )
    return f(src_r, dst_r, g, z)


def _agg_call(src_r, dst_r, g0, g1, hc):
    f = pl.kernel(
        _agg_body,
        mesh=_mesh,
        out_type=[jax.ShapeDtypeStruct((NPAD, hc), jnp.float32),
                  jax.ShapeDtypeStruct((NPAD, hc), jnp.float32)],
        scratch_types=_pipe_scratch(hc),
    )
    return f(src_r, dst_r, g0, g1)


_BM = 1024


def _tca_body(x_ref, d0_ref, d1_ref, w_ref, dinv_ref, g0_ref, g1_ref):
    dv = lax.rsqrt(d0_ref[...] + d1_ref[...] + 1.0)
    h = jnp.dot(x_ref[...], w_ref[...], preferred_element_type=jnp.float32)
    g = h * dv
    hh = g.shape[1] // 2
    dinv_ref[...] = dv
    g0_ref[...] = g[:, :hh]
    g1_ref[...] = g[:, hh:]


def _tca(x, deg0, deg1, w):
    h = w.shape[1]
    return pl.pallas_call(
        _tca_body,
        grid=(NPAD // _BM,),
        in_specs=[
            pl.BlockSpec((_BM, x.shape[1]), lambda i: (i, 0)),
            pl.BlockSpec((_BM, 1), lambda i: (i, 0)),
            pl.BlockSpec((_BM, 1), lambda i: (i, 0)),
            pl.BlockSpec(w.shape, lambda i: (0, 0)),
        ],
        out_specs=[
            pl.BlockSpec((_BM, 1), lambda i: (i, 0)),
            pl.BlockSpec((_BM, h // 2), lambda i: (i, 0)),
            pl.BlockSpec((_BM, h // 2), lambda i: (i, 0)),
        ],
        out_shape=[
            jax.ShapeDtypeStruct((NPAD, 1), jnp.float32),
            jax.ShapeDtypeStruct((NPAD, h // 2), jnp.float32),
            jax.ShapeDtypeStruct((NPAD, h // 2), jnp.float32),
        ],
    )(x, deg0, deg1, w)


def _tcb_body(a0_ref, a1_ref, dinv_ref, b_ref, w_ref, g0_ref, g1_ref):
    dv = dinv_ref[...]
    hin = w_ref.shape[0]
    hh = hin // 2
    a0 = jnp.maximum(dv * a0_ref[...] + b_ref[0, :hh], 0.0)
    a1 = jnp.maximum(dv * a1_ref[...] + b_ref[0, hh:], 0.0)
    h = (jnp.dot(a0, w_ref[:hh, :], preferred_element_type=jnp.float32)
         + jnp.dot(a1, w_ref[hh:, :], preferred_element_type=jnp.float32))
    g = h * dv
    ho = g.shape[1] // 2
    g0_ref[...] = g[:, :ho]
    g1_ref[...] = g[:, ho:]


def _tcb(acc0, acc1, dinv, b, w):
    hin, hout = w.shape
    return pl.pallas_call(
        _tcb_body,
        grid=(NPAD // _BM,),
        in_specs=[
            pl.BlockSpec((_BM, hin // 2), lambda i: (i, 0)),
            pl.BlockSpec((_BM, hin // 2), lambda i: (i, 0)),
            pl.BlockSpec((_BM, 1), lambda i: (i, 0)),
            pl.BlockSpec((1, hin), lambda i: (0, 0)),
            pl.BlockSpec((hin, hout), lambda i: (0, 0)),
        ],
        out_specs=[
            pl.BlockSpec((_BM, hout // 2), lambda i: (i, 0)),
            pl.BlockSpec((_BM, hout // 2), lambda i: (i, 0)),
        ],
        out_shape=[
            jax.ShapeDtypeStruct((NPAD, hout // 2), jnp.float32),
            jax.ShapeDtypeStruct((NPAD, hout // 2), jnp.float32),
        ],
    )(acc0, acc1, dinv, b.reshape(1, hin), w)


def _tcb_full_body(a0_ref, a1_ref, dinv_ref, b_ref, w_ref, g_ref):
    dv = dinv_ref[...]
    hin = w_ref.shape[0]
    hh = hin // 2
    a0 = jnp.maximum(dv * a0_ref[...] + b_ref[0, :hh], 0.0)
    a1 = jnp.maximum(dv * a1_ref[...] + b_ref[0, hh:], 0.0)
    h = (jnp.dot(a0, w_ref[:hh, :], preferred_element_type=jnp.float32)
         + jnp.dot(a1, w_ref[hh:, :], preferred_element_type=jnp.float32))
    g_ref[...] = h * dv


def _tcb_full(acc0, acc1, dinv, b, w):
    hin, hout = w.shape
    return pl.pallas_call(
        _tcb_full_body,
        grid=(NPAD // _BM,),
        in_specs=[
            pl.BlockSpec((_BM, hin // 2), lambda i: (i, 0)),
            pl.BlockSpec((_BM, hin // 2), lambda i: (i, 0)),
            pl.BlockSpec((_BM, 1), lambda i: (i, 0)),
            pl.BlockSpec((1, hin), lambda i: (0, 0)),
            pl.BlockSpec((hin, hout), lambda i: (0, 0)),
        ],
        out_specs=pl.BlockSpec((_BM, hout), lambda i: (i, 0)),
        out_shape=jax.ShapeDtypeStruct((NPAD, hout), jnp.float32),
    )(acc0, acc1, dinv, b.reshape(1, hin), w)


def _tcc_body(a0_ref, a1_ref, dinv_ref, b_ref, batch_ref, wc1_ref, bc1_ref,
              wc2_ref, bc2_ref, z_ref, sums_ref, cnts_ref):
    i = pl.program_id(0)
    dv = dinv_ref[...]
    a = jnp.maximum(dv * (a0_ref[...] + a1_ref[...]) + b_ref[0, :], 0.0)
    ids = lax.broadcasted_iota(jnp.int32, (_BM, B), 1)
    p = (batch_ref[...] == ids).astype(jnp.float32)

    @pl.when(i == 0)
    def _():
        sums_ref[...] = jnp.zeros_like(sums_ref)
        cnts_ref[...] = jnp.zeros_like(cnts_ref)

    sums_ref[...] += lax.dot_general(p, a, (((0,), (0,)), ((), ())),
                                     preferred_element_type=jnp.float32)
    cnts_ref[...] += lax.dot_general(p, jnp.ones((_BM, 1), jnp.float32),
                                     (((0,), (0,)), ((), ())),
                                     preferred_element_type=jnp.float32)

    @pl.when(i == pl.num_programs(0) - 1)
    def _():
        pooled = sums_ref[...] / jnp.maximum(cnts_ref[...], 1.0)
        z = jnp.maximum(jnp.dot(pooled, wc1_ref[...],
                                preferred_element_type=jnp.float32)
                        + bc1_ref[0, :], 0.0)
        z = jnp.dot(z, wc2_ref[...], preferred_element_type=jnp.float32) \
            + bc2_ref[0, :]
        z_ref[...] = 1.0 / (1.0 + jnp.exp(-z))


def _tcc(acc0, acc1, dinv, b3, batch2d, wc1, bc1, wc2, bc2):
    hin = acc0.shape[1]
    out = pl.pallas_call(
        _tcc_body,
        grid=(NPAD // _BM,),
        in_specs=[
            pl.BlockSpec((_BM, hin), lambda i: (i, 0)),
            pl.BlockSpec((_BM, hin), lambda i: (i, 0)),
            pl.BlockSpec((_BM, 1), lambda i: (i, 0)),
            pl.BlockSpec((1, hin), lambda i: (0, 0)),
            pl.BlockSpec((_BM, 1), lambda i: (i, 0)),
            pl.BlockSpec(wc1.shape, lambda i: (0, 0)),
            pl.BlockSpec((1, 32), lambda i: (0, 0)),
            pl.BlockSpec(wc2.shape, lambda i: (0, 0)),
            pl.BlockSpec((1, 1), lambda i: (0, 0)),
        ],
        out_specs=[
            pl.BlockSpec((B, 1), lambda i: (0, 0)),
            pl.BlockSpec((B, hin), lambda i: (0, 0)),
            pl.BlockSpec((B, 1), lambda i: (0, 0)),
        ],
        out_shape=[
            jax.ShapeDtypeStruct((B, 1), jnp.float32),
            jax.ShapeDtypeStruct((B, hin), jnp.float32),
            jax.ShapeDtypeStruct((B, 1), jnp.float32),
        ],
    )(acc0, acc1, dinv, b3.reshape(1, hin), batch2d,
      wc1, bc1.reshape(1, 32), wc2, bc2.reshape(1, 1))
    return out[0]


def kernel(x, edge_index, batch, W1, b1, W2, b2, W3, b3, Wc1, bc1, Wc2, bc2):
    src_r = edge_index[0]
    dst_r = edge_index[1]
    xp = jnp.pad(x, ((0, NPAD - N), (0, 0)))
    batch2d = jnp.pad(batch, (0, NPAD - N), constant_values=B).reshape(NPAD, 1)

    deg0, deg1 = _deg_call(dst_r)
    dinv, g0, g1 = _tca(xp, deg0[:, :1], deg1[:, :1], W1)

    o0, o1 = _agg_call(src_r, dst_r, g0, g1, W1.shape[1] // 2)
    g0, g1 = _tcb(o0, o1, dinv, b1, W2)

    o0, o1 = _agg_call(src_r, dst_r, g0, g1, W2.shape[1] // 2)
    g3 = _tcb_full(o0, o1, dinv, b2, W3)

    z = jnp.zeros((NPAD, W3.shape[1]), jnp.float32)
    o0, o1 = _agg2_call(src_r, dst_r, g3, z, W3.shape[1])
    return _tcc(o0, o1, dinv, b3, batch2d, Wc1, bc1, Wc2, bc2)


# speedup vs baseline: 19.0751x; 1.2620x over previous
import functools

import jax
import jax.numpy as jnp
from jax import lax
from jax.experimental import pallas as pl
from jax.experimental.pallas import tpu as pltpu
from jax.experimental.pallas import tpu_sc as plsc

N = 10000
NPAD = 10240
E = 320000
B = 64
CHUNK = 80
EROWS = E // CHUNK
TILES = 16
NODES_PER_TILE = NPAD // TILES

_mesh = plsc.VectorSubcoreMesh(core_axis_name="c", subcore_axis_name="s")


DEGW = 128


def _deg_body(dst_flat, z16, ones16, out0, out1, acc, dst_fblk, dst2d,
              ones_v, sem_x2, sem_s0, sem_s1):
    c = lax.axis_index("c")
    s = lax.axis_index("s")
    epw = E // (2 * TILES)
    tile_base = (c * TILES + s) * epw
    nblocks = epw // BLKE
    sl = pl.ds(s * NODES_PER_TILE, NODES_PER_TILE)
    sems = (sem_s0, sem_s1)

    pltpu.sync_copy(ones16, ones_v)
    pltpu.sync_copy(z16.at[sl], acc.at[sl])
    pltpu.async_copy(dst_flat.at[pl.ds(tile_base, BLKE)], dst_fblk, sem_x2)
    plsc.subcore_barrier()

    def blk_body(b, _):
        base = tile_base + b * BLKE
        pltpu.make_async_copy(
            dst_flat.at[pl.ds(base, BLKE)], dst_fblk, sem_x2).wait()
        for j in range(NBLK):
            for k in range(CHUNK // 16):
                dst2d[j, pl.ds(k * 16, 16)] = dst_fblk[
                    pl.ds(j * CHUNK + k * 16, 16)]

        @pl.when(b + 1 < nblocks)
        def _():
            pltpu.async_copy(dst_flat.at[pl.ds(base + BLKE, BLKE)],
                             dst_fblk, sem_x2)

        pends = [None, None]
        for j in range(NBLK):
            p = j & 1
            if pends[p] is not None:
                pends[p].wait()
            pends[p] = pltpu.async_copy(ones_v, acc.at[dst2d.at[j]],
                                        sems[p], add=True)
        for cp in pends:
            if cp is not None:
                cp.wait()
        return 0

    lax.fori_loop(0, nblocks, blk_body, 0)
    plsc.subcore_barrier()

    @pl.when(c == 0)
    def _():
        pltpu.sync_copy(acc.at[sl], out0.at[sl])

    @pl.when(c == 1)
    def _():
        pltpu.sync_copy(acc.at[sl], out1.at[sl])


def _deg_call(dst_r):
    z16 = jnp.zeros((NPAD, DEGW), jnp.float32)
    ones16 = jnp.ones((CHUNK, DEGW), jnp.float32)
    f = pl.kernel(
        _deg_body,
        mesh=_mesh,
        out_type=[jax.ShapeDtypeStruct((NPAD, DEGW), jnp.float32),
                  jax.ShapeDtypeStruct((NPAD, DEGW), jnp.float32)],
        scratch_types=[
            pltpu.VMEM_SHARED((NPAD, DEGW), jnp.float32),
            pltpu.VMEM((BLKE,), jnp.int32),
            pltpu.VMEM((NBLK, CHUNK), jnp.int32),
            pltpu.VMEM((CHUNK, DEGW), jnp.float32),
            pltpu.SemaphoreType.DMA,
            pltpu.SemaphoreType.DMA,
            pltpu.SemaphoreType.DMA,
        ],
    )
    return f(dst_r, z16, ones16)


AGG_NBLK = 10


def _agg_body(src_flat, dst_flat, g0, g1, o0, o1, acc, src_fblk, dst_fblk,
              src2d, dst2d, rowsn, sem_x, sem_g, sem_s):
    c = lax.axis_index("c")
    s = lax.axis_index("s")
    ept = E // TILES
    sl = pl.ds(s * NODES_PER_TILE, NODES_PER_TILE)

    def work(g_hbm, o_hbm):
        pltpu.sync_copy(g_hbm.at[sl], acc.at[sl])
        plsc.subcore_barrier()
        _edge_pipeline(s * ept, ept // (AGG_NBLK * CHUNK), AGG_NBLK, g_hbm,
                       acc, src_fblk, dst_fblk, src2d, dst2d, rowsn,
                       sem_x, sem_g, sem_s, src_flat, dst_flat)
        plsc.subcore_barrier()
        pltpu.sync_copy(acc.at[sl], o_hbm.at[sl])

    @pl.when(c == 0)
    def _():
        work(g0, o0)

    @pl.when(c == 1)
    def _():
        work(g1, o1)


NBLK = 5
BLKE = NBLK * CHUNK


NR = 4


def _edge_pipeline(tile_base, nblocks, nblk, g_hbm, acc, src_fblk, dst_fblk,
                   src2d, dst2d, rowsn, sem_x, sem_g, sem_s,
                   src_flat, dst_flat):
    blke = nblk * CHUNK

    pltpu.async_copy(src_flat.at[pl.ds(tile_base, blke)], src_fblk,
                     sem_x.at[0])
    pltpu.async_copy(dst_flat.at[pl.ds(tile_base, blke)], dst_fblk,
                     sem_x.at[1])

    def blk_body(b, _):
        base = tile_base + b * blke
        pltpu.make_async_copy(
            src_flat.at[pl.ds(base, blke)], src_fblk, sem_x.at[0]).wait()
        pltpu.make_async_copy(
            dst_flat.at[pl.ds(base, blke)], dst_fblk, sem_x.at[1]).wait()
        for j in range(nblk):
            for k in range(CHUNK // 16):
                src2d[j, pl.ds(k * 16, 16)] = src_fblk[
                    pl.ds(j * CHUNK + k * 16, 16)]
                dst2d[j, pl.ds(k * 16, 16)] = dst_fblk[
                    pl.ds(j * CHUNK + k * 16, 16)]

        @pl.when(b + 1 < nblocks)
        def _():
            nb = base + blke
            pltpu.async_copy(src_flat.at[pl.ds(nb, blke)], src_fblk,
                             sem_x.at[0])
            pltpu.async_copy(dst_flat.at[pl.ds(nb, blke)], dst_fblk,
                             sem_x.at[1])

        cps = [None] * NR
        pends = [None] * NR
        for j in range(min(2, nblk)):
            cps[j] = pltpu.async_copy(g_hbm.at[src2d.at[j]], rowsn.at[j],
                                      sem_g.at[j])
        for j in range(nblk):
            p = j % NR
            cps[p].wait()
            if j >= 2:
                pends[(j - 2) % NR].wait()
            pends[p] = pltpu.async_copy(rowsn.at[p], acc.at[dst2d.at[j]],
                                        sem_s.at[p], add=True)
            if j + 2 < nblk:
                q = (j + 2) % NR
                cps[q] = pltpu.async_copy(g_hbm.at[src2d.at[j + 2]],
                                          rowsn.at[q], sem_g.at[q])
        for t in (nblk - 2, nblk - 1):
            if t >= 0:
                pends[t % NR].wait()
        return 0

    lax.fori_loop(0, nblocks, blk_body, 0)


def _agg2_body(src_flat, dst_flat, g, z, o0, o1, acc, src_fblk, dst_fblk,
               src2d, dst2d, rowsn, sem_x, sem_g, sem_s):
    c = lax.axis_index("c")
    s = lax.axis_index("s")
    ept = E // (2 * TILES)
    sl = pl.ds(s * NODES_PER_TILE, NODES_PER_TILE)

    def work(init_hbm, o_hbm):
        pltpu.sync_copy(init_hbm.at[sl], acc.at[sl])
        plsc.subcore_barrier()
        _edge_pipeline(c * (E // 2) + s * ept, ept // BLKE, NBLK, g, acc,
                       src_fblk, dst_fblk, src2d, dst2d, rowsn,
                       sem_x, sem_g, sem_s, src_flat, dst_flat)
        plsc.subcore_barrier()
        pltpu.sync_copy(acc.at[sl], o_hbm.at[sl])

    @pl.when(c == 0)
    def _():
        work(g, o0)

    @pl.when(c == 1)
    def _():
        work(z, o1)


def _pipe_scratch(hc, nblk):
    return [
        pltpu.VMEM_SHARED((NPAD, hc), jnp.float32),
        pltpu.VMEM((nblk * CHUNK,), jnp.int32),
        pltpu.VMEM((nblk * CHUNK,), jnp.int32),
        pltpu.VMEM((nblk, CHUNK), jnp.int32),
        pltpu.VMEM((nblk, CHUNK), jnp.int32),
        pltpu.VMEM((NR, CHUNK, hc), jnp.float32),
        pltpu.SemaphoreType.DMA((2,)),
        pltpu.SemaphoreType.DMA((NR,)),
        pltpu.SemaphoreType.DMA((NR,)),
    ]


def _agg2_call(src_r, dst_r, g, z, hc):
    f = pl.kernel(
        _agg2_body,
        mesh=_mesh,
        out_type=[jax.ShapeDtypeStruct((NPAD, hc), jnp.float32),
                  jax.ShapeDtypeStruct((NPAD, hc), jnp.float32)],
        scratch_types=_pipe_scratch(hc, NBLK),
    )
    return f(src_r, dst_r, g, z)


def _agg_call(src_r, dst_r, g0, g1, hc):
    f = pl.kernel(
        _agg_body,
        mesh=_mesh,
        out_type=[jax.ShapeDtypeStruct((NPAD, hc), jnp.float32),
                  jax.ShapeDtypeStruct((NPAD, hc), jnp.float32)],
        scratch_types=_pipe_scratch(hc, AGG_NBLK),
    )
    return f(src_r, dst_r, g0, g1)


_BM = 1024


def _tca_body(x_ref, d0_ref, d1_ref, w_ref, dinv_ref, g0_ref, g1_ref):
    dv = lax.rsqrt(d0_ref[...] + d1_ref[...] + 1.0)
    h = jnp.dot(x_ref[...], w_ref[...], preferred_element_type=jnp.float32)
    g = h * dv
    hh = g.shape[1] // 2
    dinv_ref[...] = dv
    g0_ref[...] = g[:, :hh]
    g1_ref[...] = g[:, hh:]


def _tca(x, deg0, deg1, w):
    h = w.shape[1]
    return pl.pallas_call(
        _tca_body,
        grid=(NPAD // _BM,),
        in_specs=[
            pl.BlockSpec((_BM, x.shape[1]), lambda i: (i, 0)),
            pl.BlockSpec((_BM, 1), lambda i: (i, 0)),
            pl.BlockSpec((_BM, 1), lambda i: (i, 0)),
            pl.BlockSpec(w.shape, lambda i: (0, 0)),
        ],
        out_specs=[
            pl.BlockSpec((_BM, 1), lambda i: (i, 0)),
            pl.BlockSpec((_BM, h // 2), lambda i: (i, 0)),
            pl.BlockSpec((_BM, h // 2), lambda i: (i, 0)),
        ],
        out_shape=[
            jax.ShapeDtypeStruct((NPAD, 1), jnp.float32),
            jax.ShapeDtypeStruct((NPAD, h // 2), jnp.float32),
            jax.ShapeDtypeStruct((NPAD, h // 2), jnp.float32),
        ],
    )(x, deg0, deg1, w)


def _tcb_body(a0_ref, a1_ref, dinv_ref, b_ref, w_ref, g0_ref, g1_ref):
    dv = dinv_ref[...]
    hin = w_ref.shape[0]
    hh = hin // 2
    a0 = jnp.maximum(dv * a0_ref[...] + b_ref[0, :hh], 0.0)
    a1 = jnp.maximum(dv * a1_ref[...] + b_ref[0, hh:], 0.0)
    h = (jnp.dot(a0, w_ref[:hh, :], preferred_element_type=jnp.float32)
         + jnp.dot(a1, w_ref[hh:, :], preferred_element_type=jnp.float32))
    g = h * dv
    ho = g.shape[1] // 2
    g0_ref[...] = g[:, :ho]
    g1_ref[...] = g[:, ho:]


def _tcb(acc0, acc1, dinv, b, w):
    hin, hout = w.shape
    return pl.pallas_call(
        _tcb_body,
        grid=(NPAD // _BM,),
        in_specs=[
            pl.BlockSpec((_BM, hin // 2), lambda i: (i, 0)),
            pl.BlockSpec((_BM, hin // 2), lambda i: (i, 0)),
            pl.BlockSpec((_BM, 1), lambda i: (i, 0)),
            pl.BlockSpec((1, hin), lambda i: (0, 0)),
            pl.BlockSpec((hin, hout), lambda i: (0, 0)),
        ],
        out_specs=[
            pl.BlockSpec((_BM, hout // 2), lambda i: (i, 0)),
            pl.BlockSpec((_BM, hout // 2), lambda i: (i, 0)),
        ],
        out_shape=[
            jax.ShapeDtypeStruct((NPAD, hout // 2), jnp.float32),
            jax.ShapeDtypeStruct((NPAD, hout // 2), jnp.float32),
        ],
    )(acc0, acc1, dinv, b.reshape(1, hin), w)


def _tcb_full_body(a0_ref, a1_ref, dinv_ref, b_ref, w_ref, g_ref):
    dv = dinv_ref[...]
    hin = w_ref.shape[0]
    hh = hin // 2
    a0 = jnp.maximum(dv * a0_ref[...] + b_ref[0, :hh], 0.0)
    a1 = jnp.maximum(dv * a1_ref[...] + b_ref[0, hh:], 0.0)
    h = (jnp.dot(a0, w_ref[:hh, :], preferred_element_type=jnp.float32)
         + jnp.dot(a1, w_ref[hh:, :], preferred_element_type=jnp.float32))
    g_ref[...] = h * dv


def _tcb_full(acc0, acc1, dinv, b, w):
    hin, hout = w.shape
    return pl.pallas_call(
        _tcb_full_body,
        grid=(NPAD // _BM,),
        in_specs=[
            pl.BlockSpec((_BM, hin // 2), lambda i: (i, 0)),
            pl.BlockSpec((_BM, hin // 2), lambda i: (i, 0)),
            pl.BlockSpec((_BM, 1), lambda i: (i, 0)),
            pl.BlockSpec((1, hin), lambda i: (0, 0)),
            pl.BlockSpec((hin, hout), lambda i: (0, 0)),
        ],
        out_specs=pl.BlockSpec((_BM, hout), lambda i: (i, 0)),
        out_shape=jax.ShapeDtypeStruct((NPAD, hout), jnp.float32),
    )(acc0, acc1, dinv, b.reshape(1, hin), w)


def _tcc_body(a0_ref, a1_ref, dinv_ref, b_ref, batch_ref, wc1_ref, bc1_ref,
              wc2_ref, bc2_ref, z_ref, sums_ref, cnts_ref):
    i = pl.program_id(0)
    dv = dinv_ref[...]
    a = jnp.maximum(dv * (a0_ref[...] + a1_ref[...]) + b_ref[0, :], 0.0)
    ids = lax.broadcasted_iota(jnp.int32, (_BM, B), 1)
    p = (batch_ref[...] == ids).astype(jnp.float32)

    @pl.when(i == 0)
    def _():
        sums_ref[...] = jnp.zeros_like(sums_ref)
        cnts_ref[...] = jnp.zeros_like(cnts_ref)

    sums_ref[...] += lax.dot_general(p, a, (((0,), (0,)), ((), ())),
                                     preferred_element_type=jnp.float32)
    cnts_ref[...] += lax.dot_general(p, jnp.ones((_BM, 1), jnp.float32),
                                     (((0,), (0,)), ((), ())),
                                     preferred_element_type=jnp.float32)

    @pl.when(i == pl.num_programs(0) - 1)
    def _():
        pooled = sums_ref[...] / jnp.maximum(cnts_ref[...], 1.0)
        z = jnp.maximum(jnp.dot(pooled, wc1_ref[...],
                                preferred_element_type=jnp.float32)
                        + bc1_ref[0, :], 0.0)
        z = jnp.dot(z, wc2_ref[...], preferred_element_type=jnp.float32) \
            + bc2_ref[0, :]
        z_ref[...] = 1.0 / (1.0 + jnp.exp(-z))


def _tcc(acc0, acc1, dinv, b3, batch2d, wc1, bc1, wc2, bc2):
    hin = acc0.shape[1]
    out = pl.pallas_call(
        _tcc_body,
        grid=(NPAD // _BM,),
        in_specs=[
            pl.BlockSpec((_BM, hin), lambda i: (i, 0)),
            pl.BlockSpec((_BM, hin), lambda i: (i, 0)),
            pl.BlockSpec((_BM, 1), lambda i: (i, 0)),
            pl.BlockSpec((1, hin), lambda i: (0, 0)),
            pl.BlockSpec((_BM, 1), lambda i: (i, 0)),
            pl.BlockSpec(wc1.shape, lambda i: (0, 0)),
            pl.BlockSpec((1, 32), lambda i: (0, 0)),
            pl.BlockSpec(wc2.shape, lambda i: (0, 0)),
            pl.BlockSpec((1, 1), lambda i: (0, 0)),
        ],
        out_specs=[
            pl.BlockSpec((B, 1), lambda i: (0, 0)),
            pl.BlockSpec((B, hin), lambda i: (0, 0)),
            pl.BlockSpec((B, 1), lambda i: (0, 0)),
        ],
        out_shape=[
            jax.ShapeDtypeStruct((B, 1), jnp.float32),
            jax.ShapeDtypeStruct((B, hin), jnp.float32),
            jax.ShapeDtypeStruct((B, 1), jnp.float32),
        ],
    )(acc0, acc1, dinv, b3.reshape(1, hin), batch2d,
      wc1, bc1.reshape(1, 32), wc2, bc2.reshape(1, 1))
    return out[0]


def kernel(x, edge_index, batch, W1, b1, W2, b2, W3, b3, Wc1, bc1, Wc2, bc2):
    src_r = edge_index[0]
    dst_r = edge_index[1]
    xp = jnp.pad(x, ((0, NPAD - N), (0, 0)))
    batch2d = jnp.pad(batch, (0, NPAD - N), constant_values=B).reshape(NPAD, 1)

    deg0, deg1 = _deg_call(dst_r)
    dinv, g0, g1 = _tca(xp, deg0[:, :1], deg1[:, :1], W1)

    o0, o1 = _agg_call(src_r, dst_r, g0, g1, W1.shape[1] // 2)
    g0, g1 = _tcb(o0, o1, dinv, b1, W2)

    o0, o1 = _agg_call(src_r, dst_r, g0, g1, W2.shape[1] // 2)
    g3 = _tcb_full(o0, o1, dinv, b2, W3)

    z = jnp.zeros((NPAD, W3.shape[1]), jnp.float32)
    o0, o1 = _agg2_call(src_r, dst_r, g3, z, W3.shape[1])
    return _tcc(o0, o1, dinv, b3, batch2d, Wc1, bc1, Wc2, bc2)


# speedup vs baseline: 19.1065x; 1.0016x over previous
import functools

import jax
import jax.numpy as jnp
from jax import lax
from jax.experimental import pallas as pl
from jax.experimental.pallas import tpu as pltpu
from jax.experimental.pallas import tpu_sc as plsc

N = 10000
NPAD = 10240
E = 320000
B = 64
CHUNK = 80
EROWS = E // CHUNK
TILES = 16
NODES_PER_TILE = NPAD // TILES

_mesh = plsc.VectorSubcoreMesh(core_axis_name="c", subcore_axis_name="s")


DEGW = 128


DEG_NBLK = 25


def _deg_body(dst_flat, z16, ones16, out0, out1, acc, dst_fblk, dst2d,
              ones_v, sem_x2, sem_s):
    c = lax.axis_index("c")
    s = lax.axis_index("s")
    epw = E // (2 * TILES)
    tile_base = (c * TILES + s) * epw
    blke = DEG_NBLK * CHUNK
    nblocks = epw // blke
    sl = pl.ds(s * NODES_PER_TILE, NODES_PER_TILE)

    pltpu.sync_copy(ones16, ones_v)
    pltpu.sync_copy(z16.at[sl], acc.at[sl])
    pltpu.async_copy(dst_flat.at[pl.ds(tile_base, blke)], dst_fblk, sem_x2)
    plsc.subcore_barrier()

    def blk_body(b, _):
        base = tile_base + b * blke
        pltpu.make_async_copy(
            dst_flat.at[pl.ds(base, blke)], dst_fblk, sem_x2).wait()
        for j in range(DEG_NBLK):
            for k in range(CHUNK // 16):
                dst2d[j, pl.ds(k * 16, 16)] = dst_fblk[
                    pl.ds(j * CHUNK + k * 16, 16)]

        @pl.when(b + 1 < nblocks)
        def _():
            pltpu.async_copy(dst_flat.at[pl.ds(base + blke, blke)],
                             dst_fblk, sem_x2)

        pends = [None] * NR
        for j in range(DEG_NBLK):
            p = j % NR
            if pends[p] is not None:
                pends[p].wait()
            pends[p] = pltpu.async_copy(ones_v, acc.at[dst2d.at[j]],
                                        sem_s.at[p], add=True)
        for cp in pends:
            if cp is not None:
                cp.wait()
        return 0

    lax.fori_loop(0, nblocks, blk_body, 0)
    plsc.subcore_barrier()

    @pl.when(c == 0)
    def _():
        pltpu.sync_copy(acc.at[sl], out0.at[sl])

    @pl.when(c == 1)
    def _():
        pltpu.sync_copy(acc.at[sl], out1.at[sl])


def _deg_call(dst_r):
    z16 = jnp.zeros((NPAD, DEGW), jnp.float32)
    ones16 = jnp.ones((CHUNK, DEGW), jnp.float32)
    f = pl.kernel(
        _deg_body,
        mesh=_mesh,
        out_type=[jax.ShapeDtypeStruct((NPAD, DEGW), jnp.float32),
                  jax.ShapeDtypeStruct((NPAD, DEGW), jnp.float32)],
        scratch_types=[
            pltpu.VMEM_SHARED((NPAD, DEGW), jnp.float32),
            pltpu.VMEM((DEG_NBLK * CHUNK,), jnp.int32),
            pltpu.VMEM((DEG_NBLK, CHUNK), jnp.int32),
            pltpu.VMEM((CHUNK, DEGW), jnp.float32),
            pltpu.SemaphoreType.DMA,
            pltpu.SemaphoreType.DMA((NR,)),
        ],
    )
    return f(dst_r, z16, ones16)


AGG_NBLK = 10


def _agg_body(src_flat, dst_flat, g0, g1, o0, o1, acc, src_fblk, dst_fblk,
              src2d, dst2d, rowsn, sem_x, sem_g, sem_s):
    c = lax.axis_index("c")
    s = lax.axis_index("s")
    ept = E // TILES
    sl = pl.ds(s * NODES_PER_TILE, NODES_PER_TILE)

    def work(g_hbm, o_hbm):
        pltpu.sync_copy(g_hbm.at[sl], acc.at[sl])
        plsc.subcore_barrier()
        _edge_pipeline(s * ept, ept // (AGG_NBLK * CHUNK), AGG_NBLK, g_hbm,
                       acc, src_fblk, dst_fblk, src2d, dst2d, rowsn,
                       sem_x, sem_g, sem_s, src_flat, dst_flat)
        plsc.subcore_barrier()
        pltpu.sync_copy(acc.at[sl], o_hbm.at[sl])

    @pl.when(c == 0)
    def _():
        work(g0, o0)

    @pl.when(c == 1)
    def _():
        work(g1, o1)


NBLK = 5
BLKE = NBLK * CHUNK


NR = 4


def _edge_pipeline(tile_base, nblocks, nblk, g_hbm, acc, src_fblk, dst_fblk,
                   src2d, dst2d, rowsn, sem_x, sem_g, sem_s,
                   src_flat, dst_flat):
    blke = nblk * CHUNK

    pltpu.async_copy(src_flat.at[pl.ds(tile_base, blke)], src_fblk,
                     sem_x.at[0])
    pltpu.async_copy(dst_flat.at[pl.ds(tile_base, blke)], dst_fblk,
                     sem_x.at[1])

    def blk_body(b, _):
        base = tile_base + b * blke
        pltpu.make_async_copy(
            src_flat.at[pl.ds(base, blke)], src_fblk, sem_x.at[0]).wait()
        pltpu.make_async_copy(
            dst_flat.at[pl.ds(base, blke)], dst_fblk, sem_x.at[1]).wait()
        for j in range(nblk):
            for k in range(CHUNK // 16):
                src2d[j, pl.ds(k * 16, 16)] = src_fblk[
                    pl.ds(j * CHUNK + k * 16, 16)]
                dst2d[j, pl.ds(k * 16, 16)] = dst_fblk[
                    pl.ds(j * CHUNK + k * 16, 16)]

        @pl.when(b + 1 < nblocks)
        def _():
            nb = base + blke
            pltpu.async_copy(src_flat.at[pl.ds(nb, blke)], src_fblk,
                             sem_x.at[0])
            pltpu.async_copy(dst_flat.at[pl.ds(nb, blke)], dst_fblk,
                             sem_x.at[1])

        cps = [None] * NR
        pends = [None] * NR
        for j in range(min(2, nblk)):
            cps[j] = pltpu.async_copy(g_hbm.at[src2d.at[j]], rowsn.at[j],
                                      sem_g.at[j])
        for j in range(nblk):
            p = j % NR
            cps[p].wait()
            if j >= 2:
                pends[(j - 2) % NR].wait()
            pends[p] = pltpu.async_copy(rowsn.at[p], acc.at[dst2d.at[j]],
                                        sem_s.at[p], add=True)
            if j + 2 < nblk:
                q = (j + 2) % NR
                cps[q] = pltpu.async_copy(g_hbm.at[src2d.at[j + 2]],
                                          rowsn.at[q], sem_g.at[q])
        for t in (nblk - 2, nblk - 1):
            if t >= 0:
                pends[t % NR].wait()
        return 0

    lax.fori_loop(0, nblocks, blk_body, 0)


def _agg2_body(src_flat, dst_flat, g, z, o0, o1, acc, src_fblk, dst_fblk,
               src2d, dst2d, rowsn, sem_x, sem_g, sem_s):
    c = lax.axis_index("c")
    s = lax.axis_index("s")
    ept = E // (2 * TILES)
    sl = pl.ds(s * NODES_PER_TILE, NODES_PER_TILE)

    def work(init_hbm, o_hbm):
        pltpu.sync_copy(init_hbm.at[sl], acc.at[sl])
        plsc.subcore_barrier()
        _edge_pipeline(c * (E // 2) + s * ept, ept // BLKE, NBLK, g, acc,
                       src_fblk, dst_fblk, src2d, dst2d, rowsn,
                       sem_x, sem_g, sem_s, src_flat, dst_flat)
        plsc.subcore_barrier()
        pltpu.sync_copy(acc.at[sl], o_hbm.at[sl])

    @pl.when(c == 0)
    def _():
        work(g, o0)

    @pl.when(c == 1)
    def _():
        work(z, o1)


def _pipe_scratch(hc, nblk):
    return [
        pltpu.VMEM_SHARED((NPAD, hc), jnp.float32),
        pltpu.VMEM((nblk * CHUNK,), jnp.int32),
        pltpu.VMEM((nblk * CHUNK,), jnp.int32),
        pltpu.VMEM((nblk, CHUNK), jnp.int32),
        pltpu.VMEM((nblk, CHUNK), jnp.int32),
        pltpu.VMEM((NR, CHUNK, hc), jnp.float32),
        pltpu.SemaphoreType.DMA((2,)),
        pltpu.SemaphoreType.DMA((NR,)),
        pltpu.SemaphoreType.DMA((NR,)),
    ]


def _agg2_call(src_r, dst_r, g, z, hc):
    f = pl.kernel(
        _agg2_body,
        mesh=_mesh,
        out_type=[jax.ShapeDtypeStruct((NPAD, hc), jnp.float32),
                  jax.ShapeDtypeStruct((NPAD, hc), jnp.float32)],
        scratch_types=_pipe_scratch(hc, NBLK),
    )
    return f(src_r, dst_r, g, z)


def _agg_call(src_r, dst_r, g0, g1, hc):
    f = pl.kernel(
        _agg_body,
        mesh=_mesh,
        out_type=[jax.ShapeDtypeStruct((NPAD, hc), jnp.float32),
                  jax.ShapeDtypeStruct((NPAD, hc), jnp.float32)],
        scratch_types=_pipe_scratch(hc, AGG_NBLK),
    )
    return f(src_r, dst_r, g0, g1)


_BM = 1024


def _tca_body(x_ref, d0_ref, d1_ref, w_ref, dinv_ref, g0_ref, g1_ref):
    dv = lax.rsqrt(d0_ref[...] + d1_ref[...] + 1.0)
    h = jnp.dot(x_ref[...], w_ref[...], preferred_element_type=jnp.float32)
    g = h * dv
    hh = g.shape[1] // 2
    dinv_ref[...] = dv
    g0_ref[...] = g[:, :hh]
    g1_ref[...] = g[:, hh:]


def _tca(x, deg0, deg1, w):
    h = w.shape[1]
    return pl.pallas_call(
        _tca_body,
        grid=(NPAD // _BM,),
        in_specs=[
            pl.BlockSpec((_BM, x.shape[1]), lambda i: (i, 0)),
            pl.BlockSpec((_BM, 1), lambda i: (i, 0)),
            pl.BlockSpec((_BM, 1), lambda i: (i, 0)),
            pl.BlockSpec(w.shape, lambda i: (0, 0)),
        ],
        out_specs=[
            pl.BlockSpec((_BM, 1), lambda i: (i, 0)),
            pl.BlockSpec((_BM, h // 2), lambda i: (i, 0)),
            pl.BlockSpec((_BM, h // 2), lambda i: (i, 0)),
        ],
        out_shape=[
            jax.ShapeDtypeStruct((NPAD, 1), jnp.float32),
            jax.ShapeDtypeStruct((NPAD, h // 2), jnp.float32),
            jax.ShapeDtypeStruct((NPAD, h // 2), jnp.float32),
        ],
    )(x, deg0, deg1, w)


def _tcb_body(a0_ref, a1_ref, dinv_ref, b_ref, w_ref, g0_ref, g1_ref):
    dv = dinv_ref[...]
    hin = w_ref.shape[0]
    hh = hin // 2
    a0 = jnp.maximum(dv * a0_ref[...] + b_ref[0, :hh], 0.0)
    a1 = jnp.maximum(dv * a1_ref[...] + b_ref[0, hh:], 0.0)
    h = (jnp.dot(a0, w_ref[:hh, :], preferred_element_type=jnp.float32)
         + jnp.dot(a1, w_ref[hh:, :], preferred_element_type=jnp.float32))
    g = h * dv
    ho = g.shape[1] // 2
    g0_ref[...] = g[:, :ho]
    g1_ref[...] = g[:, ho:]


def _tcb(acc0, acc1, dinv, b, w):
    hin, hout = w.shape
    return pl.pallas_call(
        _tcb_body,
        grid=(NPAD // _BM,),
        in_specs=[
            pl.BlockSpec((_BM, hin // 2), lambda i: (i, 0)),
            pl.BlockSpec((_BM, hin // 2), lambda i: (i, 0)),
            pl.BlockSpec((_BM, 1), lambda i: (i, 0)),
            pl.BlockSpec((1, hin), lambda i: (0, 0)),
            pl.BlockSpec((hin, hout), lambda i: (0, 0)),
        ],
        out_specs=[
            pl.BlockSpec((_BM, hout // 2), lambda i: (i, 0)),
            pl.BlockSpec((_BM, hout // 2), lambda i: (i, 0)),
        ],
        out_shape=[
            jax.ShapeDtypeStruct((NPAD, hout // 2), jnp.float32),
            jax.ShapeDtypeStruct((NPAD, hout // 2), jnp.float32),
        ],
    )(acc0, acc1, dinv, b.reshape(1, hin), w)


def _tcb_full_body(a0_ref, a1_ref, dinv_ref, b_ref, w_ref, g_ref):
    dv = dinv_ref[...]
    hin = w_ref.shape[0]
    hh = hin // 2
    a0 = jnp.maximum(dv * a0_ref[...] + b_ref[0, :hh], 0.0)
    a1 = jnp.maximum(dv * a1_ref[...] + b_ref[0, hh:], 0.0)
    h = (jnp.dot(a0, w_ref[:hh, :], preferred_element_type=jnp.float32)
         + jnp.dot(a1, w_ref[hh:, :], preferred_element_type=jnp.float32))
    g_ref[...] = h * dv


def _tcb_full(acc0, acc1, dinv, b, w):
    hin, hout = w.shape
    return pl.pallas_call(
        _tcb_full_body,
        grid=(NPAD // _BM,),
        in_specs=[
            pl.BlockSpec((_BM, hin // 2), lambda i: (i, 0)),
            pl.BlockSpec((_BM, hin // 2), lambda i: (i, 0)),
            pl.BlockSpec((_BM, 1), lambda i: (i, 0)),
            pl.BlockSpec((1, hin), lambda i: (0, 0)),
            pl.BlockSpec((hin, hout), lambda i: (0, 0)),
        ],
        out_specs=pl.BlockSpec((_BM, hout), lambda i: (i, 0)),
        out_shape=jax.ShapeDtypeStruct((NPAD, hout), jnp.float32),
    )(acc0, acc1, dinv, b.reshape(1, hin), w)


def _tcc_body(a0_ref, a1_ref, dinv_ref, b_ref, batch_ref, wc1_ref, bc1_ref,
              wc2_ref, bc2_ref, z_ref, sums_ref, cnts_ref):
    i = pl.program_id(0)
    dv = dinv_ref[...]
    a = jnp.maximum(dv * (a0_ref[...] + a1_ref[...]) + b_ref[0, :], 0.0)
    ids = lax.broadcasted_iota(jnp.int32, (_BM, B), 1)
    p = (batch_ref[...] == ids).astype(jnp.float32)

    @pl.when(i == 0)
    def _():
        sums_ref[...] = jnp.zeros_like(sums_ref)
        cnts_ref[...] = jnp.zeros_like(cnts_ref)

    sums_ref[...] += lax.dot_general(p, a, (((0,), (0,)), ((), ())),
                                     preferred_element_type=jnp.float32)
    cnts_ref[...] += lax.dot_general(p, jnp.ones((_BM, 1), jnp.float32),
                                     (((0,), (0,)), ((), ())),
                                     preferred_element_type=jnp.float32)

    @pl.when(i == pl.num_programs(0) - 1)
    def _():
        pooled = sums_ref[...] / jnp.maximum(cnts_ref[...], 1.0)
        z = jnp.maximum(jnp.dot(pooled, wc1_ref[...],
                                preferred_element_type=jnp.float32)
                        + bc1_ref[0, :], 0.0)
        z = jnp.dot(z, wc2_ref[...], preferred_element_type=jnp.float32) \
            + bc2_ref[0, :]
        z_ref[...] = 1.0 / (1.0 + jnp.exp(-z))


def _tcc(acc0, acc1, dinv, b3, batch2d, wc1, bc1, wc2, bc2):
    hin = acc0.shape[1]
    out = pl.pallas_call(
        _tcc_body,
        grid=(NPAD // _BM,),
        in_specs=[
            pl.BlockSpec((_BM, hin), lambda i: (i, 0)),
            pl.BlockSpec((_BM, hin), lambda i: (i, 0)),
            pl.BlockSpec((_BM, 1), lambda i: (i, 0)),
            pl.BlockSpec((1, hin), lambda i: (0, 0)),
            pl.BlockSpec((_BM, 1), lambda i: (i, 0)),
            pl.BlockSpec(wc1.shape, lambda i: (0, 0)),
            pl.BlockSpec((1, 32), lambda i: (0, 0)),
            pl.BlockSpec(wc2.shape, lambda i: (0, 0)),
            pl.BlockSpec((1, 1), lambda i: (0, 0)),
        ],
        out_specs=[
            pl.BlockSpec((B, 1), lambda i: (0, 0)),
            pl.BlockSpec((B, hin), lambda i: (0, 0)),
            pl.BlockSpec((B, 1), lambda i: (0, 0)),
        ],
        out_shape=[
            jax.ShapeDtypeStruct((B, 1), jnp.float32),
            jax.ShapeDtypeStruct((B, hin), jnp.float32),
            jax.ShapeDtypeStruct((B, 1), jnp.float32),
        ],
    )(acc0, acc1, dinv, b3.reshape(1, hin), batch2d,
      wc1, bc1.reshape(1, 32), wc2, bc2.reshape(1, 1))
    return out[0]


def kernel(x, edge_index, batch, W1, b1, W2, b2, W3, b3, Wc1, bc1, Wc2, bc2):
    src_r = edge_index[0]
    dst_r = edge_index[1]
    xp = jnp.pad(x, ((0, NPAD - N), (0, 0)))
    batch2d = jnp.pad(batch, (0, NPAD - N), constant_values=B).reshape(NPAD, 1)

    deg0, deg1 = _deg_call(dst_r)
    dinv, g0, g1 = _tca(xp, deg0[:, :1], deg1[:, :1], W1)

    o0, o1 = _agg_call(src_r, dst_r, g0, g1, W1.shape[1] // 2)
    g0, g1 = _tcb(o0, o1, dinv, b1, W2)

    o0, o1 = _agg_call(src_r, dst_r, g0, g1, W2.shape[1] // 2)
    g3 = _tcb_full(o0, o1, dinv, b2, W3)

    z = jnp.zeros((NPAD, W3.shape[1]), jnp.float32)
    o0, o1 = _agg2_call(src_r, dst_r, g3, z, W3.shape[1])
    return _tcc(o0, o1, dinv, b3, batch2d, Wc1, bc1, Wc2, bc2)
